# Initial kernel scaffold; baseline (speedup 1.0000x reference)
import jax
import jax.numpy as jnp
from jax.experimental import pallas as pl


def kernel(x, edge_index, edge_attr, local_features, timestep, gamma, eta, xi, W_w, W_b, ln_g, ln_b, out_w, out_b):
    raise NotImplementedError("write your pallas kernel here")


# speedup vs baseline: 3.8760x; 3.8760x over previous
import functools

import jax
import jax.numpy as jnp
from jax import lax
from jax.experimental import pallas as pl
from jax.experimental.pallas import tpu as pltpu
from jax.experimental.pallas import tpu_sc as plsc

_NC = 2
_NS = 16
_CHUNK = 128
_G = 16


def _pre_block(x_ref, wt_ref, wb_ref, g2_ref, b2_ref, lf_ref, xi_ref,
               y_ref, i0_ref, i1_ref):
    xb = x_ref[...]
    h = jnp.dot(xb, wt_ref[...], preferred_element_type=jnp.float32)
    h = h + wb_ref[...]
    m = jnp.mean(h, axis=-1, keepdims=True)
    d = h - m
    v = jnp.mean(d * d, axis=-1, keepdims=True)
    hn = d * lax.rsqrt(v + 1e-5)
    y_ref[...] = hn * g2_ref[...] + b2_ref[...]
    i0_ref[...] = xb
    i1_ref[...] = xi_ref[0, 0] * lf_ref[...]


def _post_block(s0_ref, s1_ref, owt_ref, ob_ref, out_ref):
    u = s0_ref[...] + s1_ref[...]
    u = jnp.maximum(u, 0.0)
    out_ref[...] = (jnp.dot(u, owt_ref[...], preferred_element_type=jnp.float32)
                    + ob_ref[...] + u)


@functools.lru_cache(maxsize=None)
def _pre_call(n, d, m):
    rpt = m // _NS
    grid = (_NS,)
    row_spec = pl.BlockSpec((rpt, d), lambda i: (i, 0))
    vec_spec = pl.BlockSpec((1, d), lambda i: (0, 0))
    return pl.pallas_call(
        _pre_block,
        grid=grid,
        in_specs=[
            row_spec,
            pl.BlockSpec((d, d), lambda i: (0, 0)),
            vec_spec,
            vec_spec,
            vec_spec,
            row_spec,
            pl.BlockSpec((1, 1), lambda i: (0, 0)),
        ],
        out_specs=[row_spec, row_spec, row_spec],
        out_shape=[jax.ShapeDtypeStruct((m, d), jnp.float32)] * 3,
    )


@functools.lru_cache(maxsize=None)
def _post_call(n, d):
    blk = 1000
    grid = (n // blk,)
    row_spec = pl.BlockSpec((blk, d), lambda i: (i, 0))
    vec_spec = pl.BlockSpec((1, d), lambda i: (0, 0))
    return pl.pallas_call(
        _post_block,
        grid=grid,
        in_specs=[
            row_spec,
            row_spec,
            pl.BlockSpec((d, d), lambda i: (0, 0)),
            vec_spec,
        ],
        out_specs=row_spec,
        out_shape=jax.ShapeDtypeStruct((n, d), jnp.float32),
    )


@functools.lru_cache(maxsize=None)
def _sc_accum(m, d, k):
    rpt = m // _NS
    mesh = plsc.VectorSubcoreMesh(core_axis_name="c", subcore_axis_name="s")

    @functools.partial(
        pl.kernel, mesh=mesh,
        out_type=jax.ShapeDtypeStruct((_NC * m, d), jnp.float32),
        scratch_types=[
            pltpu.VMEM((_G, _CHUNK), jnp.int32),
            pltpu.VMEM((_G, _CHUNK), jnp.int32),
            pltpu.VMEM((_CHUNK, d), jnp.float32),
            pltpu.VMEM((_CHUNK, d), jnp.float32),
            pltpu.VMEM_SHARED((m, d), jnp.float32),
            pltpu.SemaphoreType.DMA,
            pltpu.SemaphoreType.DMA,
        ],
    )
    def body(y_hbm, i0_hbm, i1_hbm, src_hbm, dst_hbm, out_hbm,
             src_v, dst_v, rows0, rows1, s_sh, sem0, sem1):
        cid = lax.axis_index("c")
        sid = lax.axis_index("s")
        wid = sid * _NC + cid
        base = sid * rpt

        @pl.when(cid == 0)
        def _():
            pltpu.sync_copy(i0_hbm.at[pl.ds(base, rpt)],
                            s_sh.at[pl.ds(base, rpt)])

        @pl.when(cid != 0)
        def _():
            pltpu.sync_copy(i1_hbm.at[pl.ds(base, rpt)],
                            s_sh.at[pl.ds(base, rpt)])

        plsc.subcore_barrier()

        def group(g, carry):
            gbase = g * _G
            pltpu.sync_copy(src_hbm.at[wid, pl.ds(gbase, _G)], src_v)
            pltpu.sync_copy(dst_hbm.at[wid, pl.ds(gbase, _G)], dst_v)
            pltpu.async_copy(y_hbm.at[src_v.at[0]], rows0, sem0)

            def step(i, c2):
                j0 = i * 2
                j1 = j0 + 1
                pltpu.async_copy(y_hbm.at[src_v.at[j1]], rows1, sem1)
                pltpu.make_async_copy(y_hbm.at[pl.ds(0, _CHUNK)], rows0,
                                      sem0).wait()
                pltpu.sync_copy(rows0, s_sh.at[dst_v.at[j0]], add=True)

                @pl.when(j0 + 2 < _G)
                def _():
                    pltpu.async_copy(y_hbm.at[src_v.at[j0 + 2]], rows0, sem0)

                pltpu.make_async_copy(y_hbm.at[pl.ds(0, _CHUNK)], rows1,
                                      sem1).wait()
                pltpu.sync_copy(rows1, s_sh.at[dst_v.at[j1]], add=True)
                return c2

            lax.fori_loop(0, _G // 2, step, 0)
            return carry

        lax.fori_loop(0, k // _G, group, 0)
        plsc.subcore_barrier()
        pltpu.sync_copy(s_sh.at[pl.ds(base, rpt)],
                        out_hbm.at[pl.ds(cid * m + base, rpt)])

    return body


def kernel(x, edge_index, edge_attr, local_features, timestep,
           gamma, eta, xi, W_w, W_b, ln_g, ln_b, out_w, out_b):
    n, d = x.shape
    e = edge_index.shape[1]
    nw = _NC * _NS
    k = -(-e // (nw * _CHUNK * _G)) * _G
    e_pad = nw * k * _CHUNK
    m = -(-n // (_NS * 8)) * (_NS * 8)

    gamma_t = jax.nn.sigmoid(gamma[timestep])
    eta_t = jax.nn.sigmoid(eta[timestep])
    c = gamma_t * (1.0 - eta_t)
    g2 = (c * ln_g).reshape(1, d)
    b2 = (c * ln_b).reshape(1, d)

    src = edge_index[0]
    dst = edge_index[1]
    pad = e_pad - e
    src_p = jnp.concatenate(
        [src, jnp.zeros((pad,), jnp.int32)]).reshape(nw, k, _CHUNK)
    dst_p = jnp.concatenate(
        [dst, jnp.full((pad,), n, jnp.int32)]).reshape(nw, k, _CHUNK)

    y, init0, init1 = _pre_call(n, d, m)(
        x, W_w.T, W_b.reshape(1, d), g2, b2,
        local_features, xi.reshape(1, 1))

    s = _sc_accum(m, d, k)(y, init0, init1, src_p, dst_p)

    return _post_call(n, d)(
        s[:n], s[m:m + n], out_w.T, out_b.reshape(1, d))
